# baseline (speedup 1.0000x reference)
import functools

import jax
import jax.numpy as jnp
from jax import lax
from jax.experimental import pallas as pl
from jax.experimental.pallas import tpu as pltpu
from jax.experimental.pallas import tpu_sc as plsc

N_TOKENS = 8192
INPUT_SIZE = 4096
HIDDEN_SIZE = 2048
NUM_EXPERTS = 64

M_TILE = 512
_M_STEPS = N_TOKENS // M_TILE

_SC_INFO = plsc.get_sparse_core_info()
_NC = _SC_INFO.num_cores
_NS = _SC_INFO.num_subcores
_NW = _NC * _NS
_LANES = 16
_TOK_PER_W = N_TOKENS // _NW
_CHUNKS = NUM_EXPERTS // _LANES


def _router_kernel(x_ref, w1_ref, b1_ref, w2_ref, b2_ref, sel_ref):
    pre = jnp.dot(x_ref[...], w1_ref[...], preferred_element_type=jnp.float32)
    h = jnp.maximum(pre + b1_ref[...], 0.0)
    logits = jnp.dot(h, w2_ref[...], preferred_element_type=jnp.float32)
    logits = logits + b2_ref[...]
    sel_ref[0, 0, :] = jnp.argmax(logits, axis=1).astype(jnp.int32)


def _tc_router(features, W1, b1, W2, b2):
    b1r = b1.reshape(1, HIDDEN_SIZE)
    b2r = b2.reshape(1, NUM_EXPERTS)
    sel3 = pl.pallas_call(
        _router_kernel,
        grid=(_M_STEPS,),
        in_specs=[
            pl.BlockSpec((M_TILE, INPUT_SIZE), lambda m: (m, 0)),
            pl.BlockSpec((INPUT_SIZE, HIDDEN_SIZE), lambda m: (0, 0)),
            pl.BlockSpec((1, HIDDEN_SIZE), lambda m: (0, 0)),
            pl.BlockSpec((HIDDEN_SIZE, NUM_EXPERTS), lambda m: (0, 0)),
            pl.BlockSpec((1, NUM_EXPERTS), lambda m: (0, 0)),
        ],
        out_specs=pl.BlockSpec((1, 1, M_TILE), lambda m: (m, 0, 0)),
        out_shape=jax.ShapeDtypeStruct((_M_STEPS, 1, M_TILE), jnp.int32),
        compiler_params=pltpu.CompilerParams(
            dimension_semantics=("arbitrary",),
        ),
    )(features, W1, b1r, W2, b2r)
    return sel3.reshape(N_TOKENS)


def _sc_scatter_kernel(sel_hbm, out_hbm, sel_v, out_v):
    wid = lax.axis_index("s") * _NC + lax.axis_index("c")
    base = wid * _TOK_PER_W
    pltpu.sync_copy(sel_hbm.at[pl.ds(base, _TOK_PER_W)], sel_v)

    iota = lax.iota(jnp.int32, _LANES)
    gdn = lax.GatherDimensionNumbers(
        offset_dims=(), collapsed_slice_dims=(0,), start_index_map=(0,))
    splat_idx = [jnp.full((_LANES,), u, jnp.int32) for u in range(_LANES)]
    gidxs = [iota + j * _LANES for j in range(_CHUNKS)]

    def body(g, carry):
        sv = sel_v[pl.ds(g * _LANES, _LANES)]
        for u in range(_LANES):
            s = lax.gather(sv, splat_idx[u][:, None], dimension_numbers=gdn,
                           slice_sizes=(1,),
                           mode=lax.GatherScatterMode.PROMISE_IN_BOUNDS)
            t = g * _LANES + u
            for j in range(_CHUNKS):
                oh = jnp.where(gidxs[j] == s, 1.0, 0.0)
                out_v[t, pl.ds(j * _LANES, _LANES)] = oh.astype(jnp.float32)
        return carry

    lax.fori_loop(0, _TOK_PER_W // _LANES, body, 0)
    pltpu.sync_copy(out_v, out_hbm.at[pl.ds(base, _TOK_PER_W)])


@functools.partial(
    pl.kernel,
    mesh=plsc.VectorSubcoreMesh(core_axis_name="c", subcore_axis_name="s"),
    out_type=jax.ShapeDtypeStruct((N_TOKENS, NUM_EXPERTS), jnp.float32),
    scratch_types=[
        pltpu.VMEM((_TOK_PER_W,), jnp.int32),
        pltpu.VMEM((_TOK_PER_W, NUM_EXPERTS), jnp.float32),
    ],
)
def _sc_scatter(sel_hbm, out_hbm, sel_v, out_v):
    _sc_scatter_kernel(sel_hbm, out_hbm, sel_v, out_v)


@functools.partial(jax.jit, static_argnames=())
def kernel(features, W1, b1, W2, b2):
    sel = _tc_router(features, W1, b1, W2, b2)
    return _sc_scatter(sel)


# speedup vs baseline: 1.0270x; 1.0270x over previous
import functools

import jax
import jax.numpy as jnp
from jax import lax
from jax.experimental import pallas as pl
from jax.experimental.pallas import tpu as pltpu
from jax.experimental.pallas import tpu_sc as plsc

N_TOKENS = 8192
INPUT_SIZE = 4096
HIDDEN_SIZE = 2048
NUM_EXPERTS = 64

M_TILE = 512

_SC_INFO = plsc.get_sparse_core_info()
_NC = _SC_INFO.num_cores
_NS = _SC_INFO.num_subcores
_NW = _NC * _NS
_LANES = 16
_TOK_PER_W = N_TOKENS // _NW
_CHUNKS = NUM_EXPERTS // _LANES


def _logits_kernel(x_ref, w1_ref, b1_ref, w2_ref, b2_ref, out_ref):
    pre = jnp.dot(x_ref[...], w1_ref[...], preferred_element_type=jnp.float32)
    h = jnp.maximum(pre + b1_ref[...], 0.0)
    logits = jnp.dot(h, w2_ref[...], preferred_element_type=jnp.float32)
    out_ref[...] = logits + b2_ref[...]


def _tc_logits(features, W1, b1, W2, b2):
    b1r = b1.reshape(1, HIDDEN_SIZE)
    b2r = b2.reshape(1, NUM_EXPERTS)
    grid = (N_TOKENS // M_TILE,)
    return pl.pallas_call(
        _logits_kernel,
        grid=grid,
        in_specs=[
            pl.BlockSpec((M_TILE, INPUT_SIZE), lambda m: (m, 0)),
            pl.BlockSpec((INPUT_SIZE, HIDDEN_SIZE), lambda m: (0, 0)),
            pl.BlockSpec((1, HIDDEN_SIZE), lambda m: (0, 0)),
            pl.BlockSpec((HIDDEN_SIZE, NUM_EXPERTS), lambda m: (0, 0)),
            pl.BlockSpec((1, NUM_EXPERTS), lambda m: (0, 0)),
        ],
        out_specs=pl.BlockSpec((M_TILE, NUM_EXPERTS), lambda m: (m, 0)),
        out_shape=jax.ShapeDtypeStruct((N_TOKENS, NUM_EXPERTS), jnp.float32),
        compiler_params=pltpu.CompilerParams(
            dimension_semantics=("arbitrary",),
        ),
    )(features, W1, b1r, W2, b2r)


def _sc_gate_kernel(logits_hbm, out_hbm, in_v, out_v, sem0, sem1, sem_out):
    wid = lax.axis_index("s") * _NC + lax.axis_index("c")
    base = wid * _TOK_PER_W
    half = _TOK_PER_W // 2
    cin0 = pltpu.make_async_copy(
        logits_hbm.at[pl.ds(base, half)], in_v.at[pl.ds(0, half)], sem0)
    cin1 = pltpu.make_async_copy(
        logits_hbm.at[pl.ds(base + half, half)], in_v.at[pl.ds(half, half)],
        sem1)
    cin0.start()
    cin1.start()

    gdn = lax.GatherDimensionNumbers(
        offset_dims=(), collapsed_slice_dims=(0,), start_index_map=(0,))

    def permute(v, idx):
        return lax.gather(v, idx[:, None], dimension_numbers=gdn,
                          slice_sizes=(1,),
                          mode=lax.GatherScatterMode.PROMISE_IN_BOUNDS)

    iota = lax.iota(jnp.int32, _LANES)
    big = jnp.full((_LANES,), NUM_EXPERTS, jnp.int32)
    perms = [jnp.bitwise_xor(iota, sh) for sh in (8, 4, 2, 1)]
    gidxs = [iota + j * _LANES for j in range(_CHUNKS)]
    _UNROLL = 4

    def one_token(t):
        chunks = [in_v[t, pl.ds(j * _LANES, _LANES)] for j in range(_CHUNKS)]
        m = chunks[0]
        for c in chunks[1:]:
            m = jnp.maximum(m, c)
        mxv = m
        for p in perms:
            mxv = jnp.maximum(mxv, permute(mxv, p))
        cand = big
        for j in range(_CHUNKS):
            cand = jnp.minimum(cand, jnp.where(chunks[j] == mxv, gidxs[j], big))
        for p in perms:
            cand = jnp.minimum(cand, permute(cand, p))
        for j in range(_CHUNKS):
            oh = jnp.where(cand == gidxs[j], 1.0, 0.0)
            out_v[t, pl.ds(j * _LANES, _LANES)] = oh.astype(jnp.float32)

    def body(i, carry):
        for u in range(_UNROLL):
            one_token(i * _UNROLL + u)
        return carry

    steps_half = half // _UNROLL
    cin0.wait()
    lax.fori_loop(0, steps_half, body, 0)
    cout0 = pltpu.make_async_copy(
        out_v.at[pl.ds(0, half)], out_hbm.at[pl.ds(base, half)], sem_out)
    cout0.start()
    cin1.wait()
    lax.fori_loop(steps_half, 2 * steps_half, body, 0)
    pltpu.sync_copy(out_v.at[pl.ds(half, half)],
                    out_hbm.at[pl.ds(base + half, half)])
    cout0.wait()


@functools.partial(
    pl.kernel,
    mesh=plsc.VectorSubcoreMesh(core_axis_name="c", subcore_axis_name="s"),
    out_type=jax.ShapeDtypeStruct((N_TOKENS, NUM_EXPERTS), jnp.float32),
    scratch_types=[
        pltpu.VMEM((_TOK_PER_W, NUM_EXPERTS), jnp.float32),
        pltpu.VMEM((_TOK_PER_W, NUM_EXPERTS), jnp.float32),
        pltpu.SemaphoreType.DMA,
        pltpu.SemaphoreType.DMA,
        pltpu.SemaphoreType.DMA,
    ],
)
def _sc_gate(logits_hbm, out_hbm, in_v, out_v, sem0, sem1, sem_out):
    _sc_gate_kernel(logits_hbm, out_hbm, in_v, out_v, sem0, sem1, sem_out)


@functools.partial(jax.jit, static_argnames=())
def kernel(features, W1, b1, W2, b2):
    logits = _tc_logits(features, W1, b1, W2, b2)
    return _sc_gate(logits)
